# baseline (speedup 1.0000x reference)
import functools

import jax
import jax.numpy as jnp
import numpy as np
from jax import lax
from jax.experimental import pallas as pl
from jax.experimental.pallas import tpu as pltpu
from jax.experimental.pallas import tpu_sc as plsc

_IN_CH = 768
_OUT_CH = 768
_RATE = 0.95
_BATCH = 1024
_HW2 = 49
_K = int(_OUT_CH * _RATE)
_KEEP = _OUT_CH - _K
_SLICE = int(_OUT_CH * 0.75)
_NCAND = _OUT_CH - _SLICE

_CHUNK = _HW2 * 128
_NCHUNK = (_IN_CH * _HW2) // _CHUNK

_SEG = (np.arange(_CHUNK)[:, None] // _HW2 == np.arange(128)[None, :]).astype(np.float32)


def _gate_body(x_ref, seg_ref, w_ref, bias_ref, out_ref):
    a = jnp.abs(x_ref[...])
    seg = seg_ref[...].astype(jnp.bfloat16)
    acc = None
    for u in range(_NCHUNK):
        asl = a[:, u * _CHUNK:(u + 1) * _CHUNK]
        hi = asl.astype(jnp.bfloat16)
        r1 = asl - hi.astype(jnp.float32)
        mid = r1.astype(jnp.bfloat16)
        lo = (r1 - mid.astype(jnp.float32)).astype(jnp.bfloat16)
        part = (jnp.dot(hi, seg, preferred_element_type=jnp.float32)
                + jnp.dot(mid, seg, preferred_element_type=jnp.float32)
                + jnp.dot(lo, seg, preferred_element_type=jnp.float32))
        term = jnp.dot((part * np.float32(1.0 / _HW2)).astype(jnp.bfloat16),
                       w_ref[u * 128:(u + 1) * 128, :],
                       preferred_element_type=jnp.float32)
        acc = term if acc is None else acc + term
    g = jnp.maximum(acc + bias_ref[...], 0.0)
    bt = g.shape[0]
    a0 = g[:, :128].reshape(bt // 8, 1, 8, 128)
    a1 = jnp.concatenate(
        [g[:, 128:], jnp.zeros((bt, 64), jnp.float32)], axis=1
    ).reshape(bt // 8, 1, 8, 128)
    out_ref[...] = jnp.concatenate([a0, a1], axis=1).reshape(bt * 2, 128)


def _gate_candidates(xa, w_eff, bias_eff):
    bt = 128
    grid = (_BATCH // bt,)
    return pl.pallas_call(
        _gate_body,
        grid=grid,
        in_specs=[
            pl.BlockSpec((bt, _IN_CH * _HW2), lambda i: (i, 0)),
            pl.BlockSpec((_CHUNK, 128), lambda i: (0, 0)),
            pl.BlockSpec((_IN_CH, _NCAND), lambda i: (0, 0)),
            pl.BlockSpec((1, _NCAND), lambda i: (0, 0)),
        ],
        out_specs=pl.BlockSpec((bt * 2, 128), lambda i: (i, 0)),
        out_shape=jax.ShapeDtypeStruct((_BATCH * 2, 128), jnp.float32),
        compiler_params=pltpu.CompilerParams(
            dimension_semantics=("arbitrary",),
        ),
    )(xa, jnp.asarray(_SEG), w_eff, bias_eff)


def _make_sc_select():
    info = plsc.get_sparse_core_info()
    nc, ns, nl = info.num_cores, info.num_subcores, info.num_lanes
    nw = nc * ns
    rows_per_w = _BATCH // nw
    ngroups = rows_per_w // nl
    mesh = plsc.VectorSubcoreMesh(core_axis_name="c", subcore_axis_name="s")

    @functools.partial(
        pl.kernel,
        mesh=mesh,
        out_type=jax.ShapeDtypeStruct((_BATCH * _OUT_CH,), jnp.float32),
        scratch_types=[
            pltpu.VMEM((rows_per_w * 256,), jnp.float32),
            pltpu.VMEM((_NCAND * nl,), jnp.int32),
            pltpu.VMEM((_NCAND * nl,), jnp.float32),
            pltpu.VMEM((rows_per_w * _OUT_CH,), jnp.float32),
        ],
        compiler_params=pltpu.CompilerParams(needs_layout_passes=False),
    )
    def sc_select(g_hbm, out_hbm, g_v, tk_v, tt_v, out_v):
        wid = lax.axis_index("s") * nc + lax.axis_index("c")
        base = wid * rows_per_w
        pltpu.sync_copy(g_hbm.at[pl.ds(base * 256, rows_per_w * 256)], g_v)

        lane = lax.iota(jnp.int32, nl)
        zeros16 = jnp.zeros((nl,), jnp.float32)

        def zrow(r, _):
            out_v[pl.ds(pl.multiple_of(r * nl, nl), nl)] = zeros16
            return 0
        lax.fori_loop(0, rows_per_w * _OUT_CH // nl, zrow, 0)

        for gi in range(ngroups):
            row0 = gi * nl

            rl = row0 + lane
            rpart = ((lax.shift_right_logical(rl, 3) * 2048)
                     + (rl & 7) * 128)

            def tpose(c, hi):
                cpart = lax.shift_right_logical(c, 7) * 1024 + (c & 127)
                v = plsc.load_gather(g_v, [rpart + cpart])
                k = plsc.bitcast(v, jnp.int32)
                tk_v[pl.ds(pl.multiple_of(c * nl, nl), nl)] = k
                return jnp.maximum(hi, k)
            hi0 = lax.fori_loop(0, _NCAND, tpose,
                                jnp.full((nl,), -1, jnp.int32))

            def count_gt(thresh):
                def cbody(c, cnt):
                    for j in range(4):
                        k = tk_v[pl.ds(pl.multiple_of(c * 4 * nl + j * nl, nl), nl)]
                        cnt = cnt + jnp.where(k > thresh, 1, 0).astype(jnp.int32)
                    return cnt
                return lax.fori_loop(0, _NCAND // 4, cbody,
                                     jnp.zeros((nl,), jnp.int32))

            def bstep(_, carry):
                lo, hi, cnt_hi = carry
                mid = lo + lax.shift_right_arithmetic(hi - lo, 1)
                cnt = count_gt(mid)
                ge = cnt >= _KEEP
                return (jnp.where(ge, mid, lo),
                        jnp.where(ge, hi, mid),
                        jnp.where(ge, cnt_hi, cnt))
            _, thr, n_gt = lax.fori_loop(
                0, 31, bstep,
                (jnp.full((nl,), -1, jnp.int32), hi0,
                 jnp.zeros((nl,), jnp.int32)))

            fill = _KEEP - n_gt

            def sbody(i, carry):
                r, acc = carry
                c = _NCAND - 1 - i
                k = tk_v[pl.ds(pl.multiple_of(c * nl, nl), nl)]
                is_tie = k == thr
                keep = (k > thr) | (is_tie & (r < fill))
                t = jnp.where(keep, plsc.bitcast(k, jnp.float32), 0.0)
                tt_v[pl.ds(pl.multiple_of(c * nl, nl), nl)] = t
                return (r + jnp.where(is_tie, 1, 0).astype(jnp.int32),
                        acc + t)
            _, tsum = lax.fori_loop(
                0, _NCAND, sbody,
                (jnp.zeros((nl,), jnp.int32), zeros16))

            scale = _OUT_CH / (tsum + 1e-12)

            opart = ((lax.shift_right_logical(rl, 3) * 6144)
                     + (rl & 7) * 128)

            def wbody(c, _):
                co = _SLICE + c
                cpart = lax.shift_right_logical(co, 7) * 1024 + (co & 127)
                plsc.store_scatter(
                    out_v, [opart + cpart],
                    tt_v[pl.ds(pl.multiple_of(c * nl, nl), nl)] * scale)
                return 0
            lax.fori_loop(0, _NCAND, wbody, 0)

        pltpu.sync_copy(out_v, out_hbm.at[pl.ds(base * _OUT_CH, rows_per_w * _OUT_CH)])

    return sc_select


_sc_select = None


def _untile_body(in_ref, out_ref):
    x4 = in_ref[...].reshape(16, 6, 8, 128)
    parts = [x4[:, t].reshape(128, 128) for t in range(6)]
    out_ref[...] = jnp.concatenate(parts, axis=1)


def _untile(flat):
    return pl.pallas_call(
        _untile_body,
        grid=(_BATCH // 128,),
        in_specs=[pl.BlockSpec((768, 128), lambda i: (i, 0))],
        out_specs=pl.BlockSpec((128, _OUT_CH), lambda i: (i, 0)),
        out_shape=jax.ShapeDtypeStruct((_BATCH, _OUT_CH), jnp.float32),
        compiler_params=pltpu.CompilerParams(
            dimension_semantics=("arbitrary",),
        ),
    )(flat.reshape(_BATCH * _OUT_CH // 128, 128))


def kernel(x, W, b):
    global _sc_select
    if _sc_select is None:
        _sc_select = _make_sc_select()
    xa = x.reshape(_BATCH, _IN_CH * _HW2)
    w_eff = W[_SLICE:, :_IN_CH].T.astype(jnp.bfloat16)
    rate_term = (jnp.bfloat16(_RATE).astype(jnp.float32)
                 * W[_SLICE:, _IN_CH].astype(jnp.bfloat16).astype(jnp.float32))
    bias_eff = (b[_SLICE:] + rate_term)[None, :]
    g = _gate_candidates(xa, w_eff, bias_eff)
    mask_flat = _sc_select(g.reshape(-1))
    mask2d = _untile(mask_flat)
    return mask2d.reshape(_BATCH, _OUT_CH, 1, 1)


# speedup vs baseline: 4.8520x; 4.8520x over previous
import functools

import jax
import jax.numpy as jnp
import numpy as np
from jax import lax
from jax.experimental import pallas as pl
from jax.experimental.pallas import tpu as pltpu
from jax.experimental.pallas import tpu_sc as plsc

_IN_CH = 768
_OUT_CH = 768
_RATE = 0.95
_BATCH = 1024
_HW2 = 49
_K = int(_OUT_CH * _RATE)
_KEEP = _OUT_CH - _K
_SLICE = int(_OUT_CH * 0.75)
_NCAND = _OUT_CH - _SLICE


def _gate_body(x_ref, w_ref, bias_ref, out_ref):
    a = jnp.abs(x_ref[...])
    s = jnp.sum(a, axis=0) / np.float32(_HW2)
    acc = jnp.dot(s.astype(jnp.bfloat16), w_ref[...],
                  preferred_element_type=jnp.float32)
    g = jnp.maximum(acc + bias_ref[...], 0.0)
    bt = g.shape[0]
    a0 = g[:, :128].reshape(bt // 8, 1, 8, 128)
    a1 = jnp.concatenate(
        [g[:, 128:], jnp.zeros((bt, 64), jnp.float32)], axis=1
    ).reshape(bt // 8, 1, 8, 128)
    out_ref[...] = jnp.concatenate([a0, a1], axis=1).reshape(bt * 2, 128)


def _gate_candidates(xa, w_eff, bias_eff):
    bt = 128
    grid = (_BATCH // bt,)
    return pl.pallas_call(
        _gate_body,
        grid=grid,
        in_specs=[
            pl.BlockSpec((_HW2, bt, _IN_CH), lambda i: (0, i, 0)),
            pl.BlockSpec((_IN_CH, _NCAND), lambda i: (0, 0)),
            pl.BlockSpec((1, _NCAND), lambda i: (0, 0)),
        ],
        out_specs=pl.BlockSpec((bt * 2, 128), lambda i: (i, 0)),
        out_shape=jax.ShapeDtypeStruct((_BATCH * 2, 128), jnp.float32),
        compiler_params=pltpu.CompilerParams(
            dimension_semantics=("arbitrary",),
        ),
    )(xa, w_eff, bias_eff)


def _make_sc_select():
    info = plsc.get_sparse_core_info()
    nc, ns, nl = info.num_cores, info.num_subcores, info.num_lanes
    nw = nc * ns
    rows_per_w = _BATCH // nw
    ngroups = rows_per_w // nl
    mesh = plsc.VectorSubcoreMesh(core_axis_name="c", subcore_axis_name="s")

    @functools.partial(
        pl.kernel,
        mesh=mesh,
        out_type=jax.ShapeDtypeStruct((_BATCH * _OUT_CH,), jnp.float32),
        scratch_types=[
            pltpu.VMEM((rows_per_w * 256,), jnp.float32),
            pltpu.VMEM((_NCAND * nl,), jnp.int32),
            pltpu.VMEM((_NCAND * nl,), jnp.float32),
            pltpu.VMEM((rows_per_w * _OUT_CH,), jnp.float32),
        ],
        compiler_params=pltpu.CompilerParams(needs_layout_passes=False),
    )
    def sc_select(g_hbm, out_hbm, g_v, tk_v, tt_v, out_v):
        wid = lax.axis_index("s") * nc + lax.axis_index("c")
        base = wid * rows_per_w
        pltpu.sync_copy(g_hbm.at[pl.ds(base * 256, rows_per_w * 256)], g_v)

        lane = lax.iota(jnp.int32, nl)
        zeros16 = jnp.zeros((nl,), jnp.float32)

        def zrow(r, _):
            out_v[pl.ds(pl.multiple_of(r * nl, nl), nl)] = zeros16
            return 0
        lax.fori_loop(0, rows_per_w * _OUT_CH // nl, zrow, 0)

        for gi in range(ngroups):
            row0 = gi * nl

            rl = row0 + lane
            rpart = ((lax.shift_right_logical(rl, 3) * 2048)
                     + (rl & 7) * 128)

            def tpose(c, hi):
                cpart = lax.shift_right_logical(c, 7) * 1024 + (c & 127)
                v = plsc.load_gather(g_v, [rpart + cpart])
                k = plsc.bitcast(v, jnp.int32)
                tk_v[pl.ds(pl.multiple_of(c * nl, nl), nl)] = k
                return jnp.maximum(hi, k)
            hi0 = lax.fori_loop(0, _NCAND, tpose,
                                jnp.full((nl,), -1, jnp.int32))

            def count_gt(thresh):
                def cbody(c, cnt):
                    for j in range(4):
                        k = tk_v[pl.ds(pl.multiple_of(c * 4 * nl + j * nl, nl), nl)]
                        cnt = cnt + jnp.where(k > thresh, 1, 0).astype(jnp.int32)
                    return cnt
                return lax.fori_loop(0, _NCAND // 4, cbody,
                                     jnp.zeros((nl,), jnp.int32))

            def bstep(_, carry):
                lo, hi, cnt_hi = carry
                mid = lo + lax.shift_right_arithmetic(hi - lo, 1)
                cnt = count_gt(mid)
                ge = cnt >= _KEEP
                return (jnp.where(ge, mid, lo),
                        jnp.where(ge, hi, mid),
                        jnp.where(ge, cnt_hi, cnt))
            _, thr, n_gt = lax.fori_loop(
                0, 31, bstep,
                (jnp.full((nl,), -1, jnp.int32), hi0,
                 jnp.zeros((nl,), jnp.int32)))

            fill = _KEEP - n_gt

            def sbody(i, carry):
                r, acc = carry
                c = _NCAND - 1 - i
                k = tk_v[pl.ds(pl.multiple_of(c * nl, nl), nl)]
                is_tie = k == thr
                keep = (k > thr) | (is_tie & (r < fill))
                t = jnp.where(keep, plsc.bitcast(k, jnp.float32), 0.0)
                tt_v[pl.ds(pl.multiple_of(c * nl, nl), nl)] = t
                return (r + jnp.where(is_tie, 1, 0).astype(jnp.int32),
                        acc + t)
            _, tsum = lax.fori_loop(
                0, _NCAND, sbody,
                (jnp.zeros((nl,), jnp.int32), zeros16))

            scale = _OUT_CH / (tsum + 1e-12)

            opart = ((lax.shift_right_logical(rl, 3) * 6144)
                     + (rl & 7) * 128)

            def wbody(c, _):
                co = _SLICE + c
                cpart = lax.shift_right_logical(co, 7) * 1024 + (co & 127)
                plsc.store_scatter(
                    out_v, [opart + cpart],
                    tt_v[pl.ds(pl.multiple_of(c * nl, nl), nl)] * scale)
                return 0
            lax.fori_loop(0, _NCAND, wbody, 0)

        pltpu.sync_copy(out_v, out_hbm.at[pl.ds(base * _OUT_CH, rows_per_w * _OUT_CH)])

    return sc_select


_sc_select = None


def _untile_body(in_ref, out_ref):
    x4 = in_ref[...].reshape(16, 6, 8, 128)
    parts = [x4[:, t].reshape(128, 128) for t in range(6)]
    out_ref[...] = jnp.concatenate(parts, axis=1)


def _untile(flat):
    return pl.pallas_call(
        _untile_body,
        grid=(_BATCH // 128,),
        in_specs=[pl.BlockSpec((768, 128), lambda i: (i, 0))],
        out_specs=pl.BlockSpec((128, _OUT_CH), lambda i: (i, 0)),
        out_shape=jax.ShapeDtypeStruct((_BATCH, _OUT_CH), jnp.float32),
        compiler_params=pltpu.CompilerParams(
            dimension_semantics=("arbitrary",),
        ),
    )(flat.reshape(_BATCH * _OUT_CH // 128, 128))


def kernel(x, W, b):
    global _sc_select
    if _sc_select is None:
        _sc_select = _make_sc_select()
    xa = x.transpose(2, 3, 0, 1).reshape(_HW2, _BATCH, _IN_CH)
    w_eff = W[_SLICE:, :_IN_CH].T.astype(jnp.bfloat16)
    rate_term = (jnp.bfloat16(_RATE).astype(jnp.float32)
                 * W[_SLICE:, _IN_CH].astype(jnp.bfloat16).astype(jnp.float32))
    bias_eff = (b[_SLICE:] + rate_term)[None, :]
    g = _gate_candidates(xa, w_eff, bias_eff)
    mask_flat = _sc_select(g.reshape(-1))
    mask2d = _untile(mask_flat)
    return mask2d.reshape(_BATCH, _OUT_CH, 1, 1)


# speedup vs baseline: 5.2471x; 1.0814x over previous
import functools

import jax
import jax.numpy as jnp
import numpy as np
from jax import lax
from jax.experimental import pallas as pl
from jax.experimental.pallas import tpu as pltpu
from jax.experimental.pallas import tpu_sc as plsc

_IN_CH = 768
_OUT_CH = 768
_RATE = 0.95
_BATCH = 1024
_HW2 = 49
_K = int(_OUT_CH * _RATE)
_KEEP = _OUT_CH - _K
_SLICE = int(_OUT_CH * 0.75)
_NCAND = _OUT_CH - _SLICE


def _gate_body(x_ref, w_ref, bias_ref, out_ref):
    a = jnp.abs(x_ref[...])
    s = jnp.sum(a, axis=0) / np.float32(_HW2)
    acc = jnp.dot(s.astype(jnp.bfloat16), w_ref[...],
                  preferred_element_type=jnp.float32)
    g = jnp.maximum(acc + bias_ref[...], 0.0)
    bt = g.shape[0]
    a0 = g[:, :128].reshape(bt // 8, 1, 8, 128)
    a1 = jnp.concatenate(
        [g[:, 128:], jnp.zeros((bt, 64), jnp.float32)], axis=1
    ).reshape(bt // 8, 1, 8, 128)
    out_ref[...] = jnp.concatenate([a0, a1], axis=1).reshape(bt * 2, 128)


def _gate_candidates(xa, w_eff, bias_eff):
    bt = 128
    grid = (_BATCH // bt,)
    return pl.pallas_call(
        _gate_body,
        grid=grid,
        in_specs=[
            pl.BlockSpec((_HW2, bt, _IN_CH), lambda i: (0, i, 0)),
            pl.BlockSpec((_IN_CH, _NCAND), lambda i: (0, 0)),
            pl.BlockSpec((1, _NCAND), lambda i: (0, 0)),
        ],
        out_specs=pl.BlockSpec((bt * 2, 128), lambda i: (i, 0)),
        out_shape=jax.ShapeDtypeStruct((_BATCH * 2, 128), jnp.float32),
        compiler_params=pltpu.CompilerParams(
            dimension_semantics=("arbitrary",),
        ),
    )(xa, w_eff, bias_eff)


def _make_sc_select():
    info = plsc.get_sparse_core_info()
    nc, ns, nl = info.num_cores, info.num_subcores, info.num_lanes
    nw = nc * ns
    rows_per_w = _BATCH // nw
    ngroups = rows_per_w // nl
    mesh = plsc.VectorSubcoreMesh(core_axis_name="c", subcore_axis_name="s")

    @functools.partial(
        pl.kernel,
        mesh=mesh,
        out_type=jax.ShapeDtypeStruct((_BATCH * _OUT_CH,), jnp.float32),
        scratch_types=[
            pltpu.VMEM((rows_per_w * 256,), jnp.float32),
            pltpu.VMEM((_NCAND * nl,), jnp.int32),
            pltpu.VMEM((_NCAND * nl,), jnp.float32),
            pltpu.VMEM((rows_per_w * _OUT_CH,), jnp.float32),
        ],
        compiler_params=pltpu.CompilerParams(needs_layout_passes=False),
    )
    def sc_select(g_hbm, out_hbm, g_v, tk_v, tt_v, out_v):
        wid = lax.axis_index("s") * nc + lax.axis_index("c")
        base = wid * rows_per_w
        pltpu.sync_copy(g_hbm.at[pl.ds(base * 256, rows_per_w * 256)], g_v)

        lane = lax.iota(jnp.int32, nl)
        zeros16 = jnp.zeros((nl,), jnp.float32)

        def zrow(r, _):
            for j in range(8):
                out_v[pl.ds(pl.multiple_of(r * 8 * nl + j * nl, nl), nl)] = zeros16
            return 0
        lax.fori_loop(0, rows_per_w * _OUT_CH // (8 * nl), zrow, 0)

        for gi in range(ngroups):
            row0 = gi * nl

            rl = row0 + lane
            rpart = ((lax.shift_right_logical(rl, 3) * 2048)
                     + (rl & 7) * 128)

            def tpose(c, hi):
                cpart = lax.shift_right_logical(c, 7) * 1024 + (c & 127)
                v = plsc.load_gather(g_v, [rpart + cpart])
                k = plsc.bitcast(v, jnp.int32)
                tk_v[pl.ds(pl.multiple_of(c * nl, nl), nl)] = k
                return jnp.maximum(hi, k)
            hi0 = lax.fori_loop(0, _NCAND, tpose,
                                jnp.full((nl,), -1, jnp.int32))

            def count_gt(thresh):
                def cbody(c, cnt):
                    for j in range(12):
                        k = tk_v[pl.ds(pl.multiple_of(c * 12 * nl + j * nl, nl), nl)]
                        cnt = cnt + jnp.where(k > thresh, 1, 0).astype(jnp.int32)
                    return cnt
                return lax.fori_loop(0, _NCAND // 12, cbody,
                                     jnp.zeros((nl,), jnp.int32))

            def bstep(_, carry):
                lo, hi, cnt_hi = carry
                mid = lo + lax.shift_right_arithmetic(hi - lo, 1)
                cnt = count_gt(mid)
                ge = cnt >= _KEEP
                return (jnp.where(ge, mid, lo),
                        jnp.where(ge, hi, mid),
                        jnp.where(ge, cnt_hi, cnt))
            _, thr, n_gt = lax.fori_loop(
                0, 31, bstep,
                (jnp.full((nl,), -1, jnp.int32), hi0,
                 jnp.zeros((nl,), jnp.int32)))

            fill = _KEEP - n_gt

            def sbody(i, carry):
                r, acc = carry
                c = _NCAND - 1 - i
                k = tk_v[pl.ds(pl.multiple_of(c * nl, nl), nl)]
                is_tie = k == thr
                keep = (k > thr) | (is_tie & (r < fill))
                t = jnp.where(keep, plsc.bitcast(k, jnp.float32), 0.0)
                tt_v[pl.ds(pl.multiple_of(c * nl, nl), nl)] = t
                return (r + jnp.where(is_tie, 1, 0).astype(jnp.int32),
                        acc + t)
            _, tsum = lax.fori_loop(
                0, _NCAND, sbody,
                (jnp.zeros((nl,), jnp.int32), zeros16))

            scale = _OUT_CH / (tsum + 1e-12)

            opart = ((lax.shift_right_logical(rl, 3) * 6144)
                     + (rl & 7) * 128)

            def wbody(c, _):
                co = _SLICE + c
                cpart = lax.shift_right_logical(co, 7) * 1024 + (co & 127)
                plsc.store_scatter(
                    out_v, [opart + cpart],
                    tt_v[pl.ds(pl.multiple_of(c * nl, nl), nl)] * scale)
                return 0
            lax.fori_loop(0, _NCAND, wbody, 0)

        pltpu.sync_copy(out_v, out_hbm.at[pl.ds(base * _OUT_CH, rows_per_w * _OUT_CH)])

    return sc_select


_sc_select = None


def _untile_body(in_ref, out_ref):
    x4 = in_ref[...].reshape(16, 6, 8, 128)
    parts = [x4[:, t].reshape(128, 128) for t in range(6)]
    out_ref[...] = jnp.concatenate(parts, axis=1)


def _untile(flat):
    return pl.pallas_call(
        _untile_body,
        grid=(_BATCH // 128,),
        in_specs=[pl.BlockSpec((768, 128), lambda i: (i, 0))],
        out_specs=pl.BlockSpec((128, _OUT_CH), lambda i: (i, 0)),
        out_shape=jax.ShapeDtypeStruct((_BATCH, _OUT_CH), jnp.float32),
        compiler_params=pltpu.CompilerParams(
            dimension_semantics=("arbitrary",),
        ),
    )(flat.reshape(_BATCH * _OUT_CH // 128, 128))


def kernel(x, W, b):
    global _sc_select
    if _sc_select is None:
        _sc_select = _make_sc_select()
    xa = x.transpose(2, 3, 0, 1).reshape(_HW2, _BATCH, _IN_CH)
    w_eff = W[_SLICE:, :_IN_CH].T.astype(jnp.bfloat16)
    rate_term = (jnp.bfloat16(_RATE).astype(jnp.float32)
                 * W[_SLICE:, _IN_CH].astype(jnp.bfloat16).astype(jnp.float32))
    bias_eff = (b[_SLICE:] + rate_term)[None, :]
    g = _gate_candidates(xa, w_eff, bias_eff)
    mask_flat = _sc_select(g.reshape(-1))
    mask2d = _untile(mask_flat)
    return mask2d.reshape(_BATCH, _OUT_CH, 1, 1)
